# baseline (device time: 44001 ns/iter reference)
import jax
import jax.numpy as jnp
from jax import lax
from jax.experimental import pallas as pl
from jax.experimental.pallas import tpu as pltpu

N_DEV = 4
B, SQ, SKV, D = 2, 256, 512, 768
H, DH = 8, 64
R = B * SQ
HALF, QTR = R // 2, R // 4


def kernel(x, Wq, Wo, K_ext, V_ext):
    def body(x_ref, wq_ref, wo_ref, k_ref, v_ref, out_ref,
             acc_ref, attn_ref, sbuf_a, rbuf_a, sbuf_b, rbuf_b,
             sbuf_b2, rbuf_b2, sbuf_a2, rbuf_a2, send_sems, recv_sems):
        my = lax.axis_index("i")
        peer_a = my ^ 1
        peer_b = 3 - my

        barrier_sem = pltpu.get_barrier_semaphore()
        for nbr in (peer_a, peer_b):
            pl.semaphore_signal(
                barrier_sem, inc=1,
                device_id=(nbr,), device_id_type=pl.DeviceIdType.MESH,
            )
        pl.semaphore_wait(barrier_sem, 2)

        wq_bf = wq_ref[...].astype(jnp.bfloat16)
        wo_bf = wo_ref[...].astype(jnp.bfloat16)
        for b in range(B):
            q = jnp.dot(x_ref[b].astype(jnp.bfloat16), wq_bf,
                        preferred_element_type=jnp.float32)
            q_bf = q.astype(jnp.bfloat16)
            for h in range(H):
                qh = q_bf[:, h * DH:(h + 1) * DH]
                kh = k_ref[b, :, h, :].astype(jnp.bfloat16)
                vh = v_ref[b, :, h, :].astype(jnp.bfloat16)
                s = lax.dot_general(
                    qh, kh, (((1,), (1,)), ((), ())),
                    preferred_element_type=jnp.float32) * 0.125
                m = jnp.max(s, axis=1, keepdims=True)
                p = jnp.exp(s - m)
                l = jnp.sum(p, axis=1, keepdims=True)
                o = jnp.dot(p.astype(jnp.bfloat16), vh,
                            preferred_element_type=jnp.float32) / l
                attn_ref[:, h * DH:(h + 1) * DH] = o.astype(jnp.bfloat16)
            acc_ref[b * SQ:(b + 1) * SQ, :] = jnp.dot(
                attn_ref[...], wo_bf,
                preferred_element_type=jnp.float32)

        is03 = (my == 0) | (my == 3)
        half_keep = jnp.where(is03, 0, HALF)
        half_send = HALF - half_keep
        q_add = jnp.where(my <= 1, 0, QTR)
        q_keep = half_keep + q_add
        q_send = half_keep + (QTR - q_add)

        def xchg(src, dst, idx, peer):
            rdma = pltpu.make_async_remote_copy(
                src_ref=src, dst_ref=dst,
                send_sem=send_sems.at[idx], recv_sem=recv_sems.at[idx],
                device_id=(peer,), device_id_type=pl.DeviceIdType.MESH,
            )
            rdma.start()
            rdma.wait()


        sbuf_a[...] = acc_ref[pl.ds(half_send, HALF)].astype(jnp.bfloat16)
        xchg(sbuf_a, rbuf_a, 0, peer_a)
        acc_ref[pl.ds(half_keep, HALF)] = (
            acc_ref[pl.ds(half_keep, HALF)] + rbuf_a[...].astype(jnp.float32))

        sbuf_b[...] = acc_ref[pl.ds(q_send, QTR)].astype(jnp.bfloat16)
        xchg(sbuf_b, rbuf_b, 1, peer_b)
        acc_ref[pl.ds(q_keep, QTR)] = (
            acc_ref[pl.ds(q_keep, QTR)] + rbuf_b[...].astype(jnp.float32))

        sbuf_b2[...] = acc_ref[pl.ds(q_keep, QTR)].astype(jnp.bfloat16)
        xchg(sbuf_b2, rbuf_b2, 2, peer_b)
        acc_ref[pl.ds(q_send, QTR)] = rbuf_b2[...].astype(jnp.float32)

        sbuf_a2[...] = acc_ref[pl.ds(half_keep, HALF)].astype(jnp.bfloat16)
        xchg(sbuf_a2, rbuf_a2, 3, peer_a)
        acc_ref[pl.ds(half_send, HALF)] = rbuf_a2[...].astype(jnp.float32)

        out_ref[0] = acc_ref[0:SQ, :]
        out_ref[1] = acc_ref[SQ:R, :]

    return pl.pallas_call(
        body,
        out_shape=jax.ShapeDtypeStruct((B, SQ, D), jnp.float32),
        in_specs=[pl.BlockSpec(memory_space=pltpu.VMEM)] * 5,
        out_specs=pl.BlockSpec(memory_space=pltpu.VMEM),
        scratch_shapes=[
            pltpu.VMEM((R, D), jnp.float32),
            pltpu.VMEM((SQ, H * DH), jnp.bfloat16),
            pltpu.VMEM((HALF, D), jnp.bfloat16),
            pltpu.VMEM((HALF, D), jnp.bfloat16),
            pltpu.VMEM((QTR, D), jnp.bfloat16),
            pltpu.VMEM((QTR, D), jnp.bfloat16),
            pltpu.VMEM((QTR, D), jnp.bfloat16),
            pltpu.VMEM((QTR, D), jnp.bfloat16),
            pltpu.VMEM((HALF, D), jnp.bfloat16),
            pltpu.VMEM((HALF, D), jnp.bfloat16),
            pltpu.SemaphoreType.DMA((4,)),
            pltpu.SemaphoreType.DMA((4,)),
        ],
        compiler_params=pltpu.CompilerParams(collective_id=0),
    )(x, Wq, Wo, K_ext, V_ext)


# device time: 34413 ns/iter; 1.2786x vs baseline; 1.2786x over previous
import jax
import jax.numpy as jnp
from jax import lax
from jax.experimental import pallas as pl
from jax.experimental.pallas import tpu as pltpu

N_DEV = 4
B, SQ, SKV, D = 2, 256, 512, 768
H, DH = 8, 64
R = B * SQ
HALF, QTR = R // 2, R // 4


def kernel(x, Wq, Wo, K_ext, V_ext):
    def body(x_ref, wq_ref, wo_ref, k_ref, v_ref, out_ref,
             acc_ref, attn_ref, sbuf_a, rbuf_a, sbuf_b, rbuf_b,
             sbuf_b2, rbuf_b2, sbuf_a2, rbuf_a2, send_sems, recv_sems):
        my = lax.axis_index("i")
        peer_a = my ^ 1
        peer_b = 3 - my

        barrier_sem = pltpu.get_barrier_semaphore()
        for nbr in (peer_a, peer_b):
            pl.semaphore_signal(
                barrier_sem, inc=1,
                device_id=(nbr,), device_id_type=pl.DeviceIdType.MESH,
            )
        pl.semaphore_wait(barrier_sem, 2)

        is03 = (my == 0) | (my == 3)
        half_keep = jnp.where(is03, 0, HALF)
        half_send = HALF - half_keep
        q_add = jnp.where(my <= 1, 0, QTR)
        q_keep = half_keep + q_add
        q_send = half_keep + (QTR - q_add)

        def compute_batch(b):
            q = jnp.dot(x_ref[b], wq_ref[...],
                        preferred_element_type=jnp.float32)
            for h in range(H):
                qh = q[:, h * DH:(h + 1) * DH]
                kh = k_ref[b, :, h, :]
                vh = v_ref[b, :, h, :]
                s = lax.dot_general(
                    qh, kh, (((1,), (1,)), ((), ())),
                    preferred_element_type=jnp.float32) * 0.125
                m = jnp.max(s, axis=1, keepdims=True)
                p = jnp.exp(s - m)
                l = jnp.sum(p, axis=1, keepdims=True)
                o = jnp.dot(p, vh, preferred_element_type=jnp.float32) / l
                attn_ref[:, h * DH:(h + 1) * DH] = o
            acc_ref[b * SQ:(b + 1) * SQ, :] = jnp.dot(
                attn_ref[...], wo_ref[...],
                preferred_element_type=jnp.float32)

        send_b1 = half_send == HALF
        pl.when(send_b1)(lambda: compute_batch(1))
        pl.when(jnp.logical_not(send_b1))(lambda: compute_batch(0))

        sbuf_a[...] = acc_ref[pl.ds(half_send, HALF)].astype(jnp.bfloat16)
        rdma_a = pltpu.make_async_remote_copy(
            src_ref=sbuf_a, dst_ref=rbuf_a,
            send_sem=send_sems.at[0], recv_sem=recv_sems.at[0],
            device_id=(peer_a,), device_id_type=pl.DeviceIdType.MESH,
        )
        rdma_a.start()

        pl.when(send_b1)(lambda: compute_batch(0))
        pl.when(jnp.logical_not(send_b1))(lambda: compute_batch(1))

        def xchg(src, dst, idx, peer):
            rdma = pltpu.make_async_remote_copy(
                src_ref=src, dst_ref=dst,
                send_sem=send_sems.at[idx], recv_sem=recv_sems.at[idx],
                device_id=(peer,), device_id_type=pl.DeviceIdType.MESH,
            )
            rdma.start()
            rdma.wait()


        rdma_a.wait()
        acc_ref[pl.ds(half_keep, HALF)] = (
            acc_ref[pl.ds(half_keep, HALF)] + rbuf_a[...].astype(jnp.float32))

        sbuf_b[...] = acc_ref[pl.ds(q_send, QTR)].astype(jnp.bfloat16)
        xchg(sbuf_b, rbuf_b, 1, peer_b)
        acc_ref[pl.ds(q_keep, QTR)] = (
            acc_ref[pl.ds(q_keep, QTR)] + rbuf_b[...].astype(jnp.float32))

        sbuf_b2[...] = acc_ref[pl.ds(q_keep, QTR)].astype(jnp.bfloat16)
        xchg(sbuf_b2, rbuf_b2, 2, peer_b)
        acc_ref[pl.ds(q_send, QTR)] = rbuf_b2[...].astype(jnp.float32)

        sbuf_a2[...] = acc_ref[pl.ds(half_keep, HALF)].astype(jnp.bfloat16)
        xchg(sbuf_a2, rbuf_a2, 3, peer_a)
        acc_ref[pl.ds(half_send, HALF)] = rbuf_a2[...].astype(jnp.float32)

        out_ref[0] = acc_ref[0:SQ, :]
        out_ref[1] = acc_ref[SQ:R, :]

    return pl.pallas_call(
        body,
        out_shape=jax.ShapeDtypeStruct((B, SQ, D), jnp.float32),
        in_specs=[pl.BlockSpec(memory_space=pltpu.VMEM)] * 5,
        out_specs=pl.BlockSpec(memory_space=pltpu.VMEM),
        scratch_shapes=[
            pltpu.VMEM((R, D), jnp.float32),
            pltpu.VMEM((SQ, H * DH), jnp.float32),
            pltpu.VMEM((HALF, D), jnp.bfloat16),
            pltpu.VMEM((HALF, D), jnp.bfloat16),
            pltpu.VMEM((QTR, D), jnp.bfloat16),
            pltpu.VMEM((QTR, D), jnp.bfloat16),
            pltpu.VMEM((QTR, D), jnp.bfloat16),
            pltpu.VMEM((QTR, D), jnp.bfloat16),
            pltpu.VMEM((HALF, D), jnp.bfloat16),
            pltpu.VMEM((HALF, D), jnp.bfloat16),
            pltpu.SemaphoreType.DMA((4,)),
            pltpu.SemaphoreType.DMA((4,)),
        ],
        compiler_params=pltpu.CompilerParams(collective_id=0),
    )(x, Wq, Wo, K_ext, V_ext)


# device time: 30870 ns/iter; 1.4254x vs baseline; 1.1148x over previous
import os

import jax
import jax.numpy as jnp
from jax import lax
from jax.experimental import pallas as pl
from jax.experimental.pallas import tpu as pltpu

N_DEV = 4
B, SQ, SKV, D = 2, 256, 512, 768
H, DH = 8, 64
R = B * SQ
HALF, QTR = R // 2, R // 4
_PROBE = os.environ.get("KPROBE", "")


def kernel(x, Wq, Wo, K_ext, V_ext):
    def body(x_ref, wq_ref, wo_ref, k_ref, v_ref, out_ref,
             acc_ref, attn_ref, sbuf_a, rbuf_a, sbuf_b, rbuf_b,
             sbuf_q, gbuf, send_sems, recv_sems,
             ag_send_sems, ag_recv_sems):
        my = lax.axis_index("i")
        peer_a = my ^ 1
        peer_b = 3 - my

        barrier_sem = pltpu.get_barrier_semaphore()
        for nbr in (peer_a, peer_b):
            pl.semaphore_signal(
                barrier_sem, inc=1,
                device_id=(nbr,), device_id_type=pl.DeviceIdType.MESH,
            )
        pl.semaphore_wait(barrier_sem, 2)

        is03 = (my == 0) | (my == 3)
        half_keep = jnp.where(is03, 0, HALF)
        half_send = HALF - half_keep
        q_add = jnp.where(my <= 1, 0, QTR)
        q_keep = half_keep + q_add
        q_send = half_keep + (QTR - q_add)

        def compute_batch(b):
            q = jnp.dot(x_ref[b], wq_ref[...],
                        preferred_element_type=jnp.float32)
            for h in range(H):
                qh = q[:, h * DH:(h + 1) * DH]
                kh = k_ref[b, :, h, :]
                vh = v_ref[b, :, h, :]
                s = lax.dot_general(
                    qh, kh, (((1,), (1,)), ((), ())),
                    preferred_element_type=jnp.float32) * 0.125
                m = jnp.max(s, axis=1, keepdims=True)
                p = jnp.exp(s - m)
                l = jnp.sum(p, axis=1, keepdims=True)
                o = jnp.dot(p, vh, preferred_element_type=jnp.float32) / l
                attn_ref[:, h * DH:(h + 1) * DH] = o
            acc_ref[b * SQ:(b + 1) * SQ, :] = jnp.dot(
                attn_ref[...], wo_ref[...],
                preferred_element_type=jnp.float32)

        send_b1 = half_send == HALF
        if _PROBE != "comm":
            pl.when(send_b1)(lambda: compute_batch(1))
            pl.when(jnp.logical_not(send_b1))(lambda: compute_batch(0))

        if _PROBE != "compute":
            sbuf_a[...] = acc_ref[pl.ds(half_send, HALF)].astype(jnp.bfloat16)
            rdma_a = pltpu.make_async_remote_copy(
                src_ref=sbuf_a, dst_ref=rbuf_a,
                send_sem=send_sems.at[0], recv_sem=recv_sems.at[0],
                device_id=(peer_a,), device_id_type=pl.DeviceIdType.MESH,
            )
            rdma_a.start()

        if _PROBE != "comm":
            pl.when(send_b1)(lambda: compute_batch(0))
            pl.when(jnp.logical_not(send_b1))(lambda: compute_batch(1))

        def xchg(src, dst, idx, peer):
            rdma = pltpu.make_async_remote_copy(
                src_ref=src, dst_ref=dst,
                send_sem=send_sems.at[idx], recv_sem=recv_sems.at[idx],
                device_id=(peer,), device_id_type=pl.DeviceIdType.MESH,
            )
            rdma.start()
            rdma.wait()


        if _PROBE != "compute":
            rdma_a.wait()
            acc_ref[pl.ds(half_keep, HALF)] = (
                acc_ref[pl.ds(half_keep, HALF)]
                + rbuf_a[...].astype(jnp.float32))

            sbuf_b[...] = acc_ref[pl.ds(q_send, QTR)].astype(jnp.bfloat16)
            xchg(sbuf_b, rbuf_b, 1, peer_b)
            acc_ref[pl.ds(q_keep, QTR)] = (
                acc_ref[pl.ds(q_keep, QTR)]
                + rbuf_b[...].astype(jnp.float32))

            def qkeep_of(d):
                hk = jnp.where((d == 0) | (d == 3), 0, HALF)
                return hk + jnp.where(d <= 1, 0, QTR)

            def out_write(rows_lo, val):
                b_i = jnp.where(rows_lo >= SQ, 1, 0)
                r0 = rows_lo - b_i * SQ
                out_ref[pl.ds(b_i, 1), pl.ds(r0, QTR), :] = (
                    val.reshape(1, QTR, D))

            sbuf_q[...] = acc_ref[pl.ds(q_keep, QTR)].astype(jnp.bfloat16)
            sends = []
            for r in range(3):
                peer_r = lax.rem(my + 1 + r, N_DEV)
                rdma = pltpu.make_async_remote_copy(
                    src_ref=sbuf_q, dst_ref=gbuf.at[2 - r],
                    send_sem=ag_send_sems.at[r],
                    recv_sem=ag_recv_sems.at[2 - r],
                    device_id=(peer_r,), device_id_type=pl.DeviceIdType.MESH,
                )
                rdma.start()
                sends.append(rdma)

            out_write(q_keep, acc_ref[pl.ds(q_keep, QTR)])

            for s in range(3):
                recv = pltpu.make_async_remote_copy(
                    src_ref=gbuf.at[s], dst_ref=gbuf.at[s],
                    send_sem=ag_send_sems.at[s],
                    recv_sem=ag_recv_sems.at[s],
                    device_id=(my,), device_id_type=pl.DeviceIdType.MESH,
                )
                recv.wait_recv()
                out_write(qkeep_of(lax.rem(my + 3 - s, N_DEV)),
                          gbuf[s].astype(jnp.float32))

            for rdma in sends:
                rdma.wait_send()

    return pl.pallas_call(
        body,
        out_shape=jax.ShapeDtypeStruct((B, SQ, D), jnp.float32),
        in_specs=[pl.BlockSpec(memory_space=pltpu.VMEM)] * 5,
        out_specs=pl.BlockSpec(memory_space=pltpu.VMEM),
        scratch_shapes=[
            pltpu.VMEM((R, D), jnp.float32),
            pltpu.VMEM((SQ, H * DH), jnp.float32),
            pltpu.VMEM((HALF, D), jnp.bfloat16),
            pltpu.VMEM((HALF, D), jnp.bfloat16),
            pltpu.VMEM((QTR, D), jnp.bfloat16),
            pltpu.VMEM((QTR, D), jnp.bfloat16),
            pltpu.VMEM((QTR, D), jnp.bfloat16),
            pltpu.VMEM((3, QTR, D), jnp.bfloat16),
            pltpu.SemaphoreType.DMA((2,)),
            pltpu.SemaphoreType.DMA((2,)),
            pltpu.SemaphoreType.DMA((3,)),
            pltpu.SemaphoreType.DMA((3,)),
        ],
        compiler_params=pltpu.CompilerParams(collective_id=0),
    )(x, Wq, Wo, K_ext, V_ext)
